# baseline (speedup 1.0000x reference)
import functools

import jax
import jax.numpy as jnp
from jax import lax
from jax.experimental import pallas as pl
from jax.experimental.pallas import tpu as pltpu
from jax.experimental.pallas import tpu_sc as plsc

NC = 2
NS = 16
NW = NC * NS
RING = 6


@functools.cache
def _make_sorted_gather(v, d, n):
  cpw = n // NW
  mesh = plsc.VectorSubcoreMesh(core_axis_name="c", subcore_axis_name="s")

  def body(idx_hbm, tabt_hbm, out_hbm, idxv, gring, ostage, *gsem):
    wid = lax.axis_index("s") * NC + lax.axis_index("c")
    base = wid * cpw
    iota = lax.broadcasted_iota(jnp.int32, (16,), 0)

    pltpu.sync_copy(idx_hbm.at[pl.ds(base, cpw)], idxv.at[pl.ds(0, cpw)])
    idxv[pl.ds(cpw, 16)] = jnp.full((16,), -1, dtype=jnp.int32)

    dnums = lax.GatherDimensionNumbers(
        offset_dims=(), collapsed_slice_dims=(0,), start_index_map=(0,))

    def at(ptr):
      vb = idxv[pl.ds((ptr >> 4) << 4, 16)]
      sp = lax.gather(
          vb, jnp.full((16, 1), ptr & 15, dtype=jnp.int32), dnums,
          slice_sizes=(1,), mode=lax.GatherScatterMode.PROMISE_IN_BOUNDS)
      return sp[0]

    def fetch(c, slot):
      col = pl.multiple_of(c * 128, 128)
      pltpu.async_copy(
          tabt_hbm.at[:, pl.ds(col, 128)], gring.at[slot], gsem[slot])

    c_lo = idxv[pl.ds(0, 16)][0] >> 7
    c_hi = at(cpw - 1) >> 7
    ncols = c_hi - c_lo + 1

    for r in range(RING):
      @pl.when(r < ncols)
      def _():
        fetch(c_lo + r, r)

    def col_step(g, ptr0):
      ptr = ptr0
      for r in range(RING):
        p = g * RING + r

        @pl.when(p < ncols)
        def _():
          pltpu.make_async_copy(
              tabt_hbm.at[:, pl.ds(0, 128)], gring.at[r], gsem[r]).wait()

        c = c_lo + p

        def w_cond(ptr):
          return (p < ncols) & ((at(ptr) >> 7) == c)

        def w_body(ptr):
          lane = jnp.full((16,), at(ptr) & 127, dtype=jnp.int32)
          for q in range(d // 16):
            vals = plsc.load_gather(gring.at[r], [iota + 16 * q, lane])
            ostage[ptr >> 3, ptr & 7, pl.ds(16 * q, 16)] = vals
          return ptr + 1

        ptr = lax.while_loop(w_cond, w_body, ptr)

        @pl.when(p + RING < ncols)
        def _():
          fetch(c_lo + p + RING, r)

      return ptr

    lax.fori_loop(0, (ncols + RING - 1) // RING, col_step, 0)
    pltpu.sync_copy(ostage, out_hbm.at[pl.ds(base // 8, cpw // 8)])

  return pl.kernel(
      body,
      out_type=jax.ShapeDtypeStruct((n // 8, 8, d), jnp.float32),
      mesh=mesh,
      scratch_types=[
          pltpu.VMEM((cpw + 16,), jnp.int32),
          pltpu.VMEM((RING, d, 128), jnp.float32),
          pltpu.VMEM((cpw // 8, 8, d), jnp.float32),
      ] + [pltpu.SemaphoreType.DMA] * RING,
      compiler_params=pltpu.CompilerParams(needs_layout_passes=False),
  )


@functools.cache
def _make_unpermute(d, n):
  cpw = n // NW
  mesh = plsc.VectorSubcoreMesh(core_axis_name="c", subcore_axis_name="s")

  def body(inv_hbm, src_hbm, out_hbm, idxv, ostage, sem):
    wid = lax.axis_index("s") * NC + lax.axis_index("c")
    base = wid * cpw

    pltpu.sync_copy(inv_hbm.at[pl.ds(base, cpw)], idxv)

    def fire(ch, carry):
      vec = idxv[pl.ds(ch * 16, 16)]
      for l in range(16):
        i = vec[l]
        pltpu.async_copy(
            src_hbm.at[i >> 3, i & 7], ostage.at[ch * 2 + l // 8, l % 8], sem)
      return carry

    lax.fori_loop(0, cpw // 16, fire, 0)

    def drain(k, carry):
      pltpu.make_async_copy(
          src_hbm.at[0, 0], ostage.at[k // 8, k % 8], sem).wait()
      return carry

    lax.fori_loop(0, cpw, drain, 0)
    pltpu.sync_copy(ostage, out_hbm.at[pl.ds(base // 8, cpw // 8)])

  return pl.kernel(
      body,
      out_type=jax.ShapeDtypeStruct((n // 8, 8, d), jnp.float32),
      mesh=mesh,
      scratch_types=[
          pltpu.VMEM((cpw,), jnp.int32),
          pltpu.VMEM((cpw // 8, 8, d), jnp.float32),
          pltpu.SemaphoreType.DMA,
      ],
  )


def kernel(user_indices, embedding_table):
  (n,) = user_indices.shape
  v, d = embedding_table.shape
  idx = user_indices.astype(jnp.int32)
  ar = jnp.arange(n, dtype=jnp.int32)
  idx_s, perm = lax.sort((idx, ar), num_keys=1)
  _, inv = lax.sort((perm, ar), num_keys=1)
  outs = _make_sorted_gather(v, d, n)(idx_s, embedding_table.T)
  out3 = _make_unpermute(d, n)(inv, outs)
  return out3.reshape(n, d)


# speedup vs baseline: 1.0128x; 1.0128x over previous
import functools

import jax
import jax.numpy as jnp
from jax import lax
from jax.experimental import pallas as pl
from jax.experimental.pallas import tpu as pltpu
from jax.experimental.pallas import tpu_sc as plsc

NC = 2
NS = 16
NW = NC * NS
RING = 7


@functools.cache
def _make_sorted_gather(v, d, n):
  cpw = n // NW
  mesh = plsc.VectorSubcoreMesh(core_axis_name="c", subcore_axis_name="s")

  def body(idx_hbm, tabt_hbm, out_hbm, idxv, gring, ostage, *gsem):
    wid = lax.axis_index("s") * NC + lax.axis_index("c")
    base = wid * cpw
    iota = lax.broadcasted_iota(jnp.int32, (16,), 0)

    pltpu.sync_copy(idx_hbm.at[pl.ds(base, cpw)], idxv.at[pl.ds(0, cpw)])
    idxv[pl.ds(cpw, 16)] = jnp.full((16,), -1, dtype=jnp.int32)

    dnums = lax.GatherDimensionNumbers(
        offset_dims=(), collapsed_slice_dims=(0,), start_index_map=(0,))

    def at(ptr):
      vb = idxv[pl.ds((ptr >> 4) << 4, 16)]
      sp = lax.gather(
          vb, jnp.full((16, 1), ptr & 15, dtype=jnp.int32), dnums,
          slice_sizes=(1,), mode=lax.GatherScatterMode.PROMISE_IN_BOUNDS)
      return sp[0]

    def fetch(c, slot):
      col = pl.multiple_of(c * 128, 128)
      pltpu.async_copy(
          tabt_hbm.at[:, pl.ds(col, 128)], gring.at[slot], gsem[slot])

    c_lo = idxv[pl.ds(0, 16)][0] >> 7
    c_hi = at(cpw - 1) >> 7
    ncols = c_hi - c_lo + 1

    for r in range(RING):
      @pl.when(r < ncols)
      def _():
        fetch(c_lo + r, r)

    def col_step(g, ptr0):
      ptr = ptr0
      for r in range(RING):
        p = g * RING + r

        @pl.when(p < ncols)
        def _():
          pltpu.make_async_copy(
              tabt_hbm.at[:, pl.ds(0, 128)], gring.at[r], gsem[r]).wait()

        c = c_lo + p

        def w_cond(ptr):
          return (p < ncols) & ((at(ptr) >> 7) == c)

        def w_body(ptr):
          lane = jnp.full((16,), at(ptr) & 127, dtype=jnp.int32)
          for q in range(d // 16):
            vals = plsc.load_gather(gring.at[r], [iota + 16 * q, lane])
            ostage[ptr >> 3, ptr & 7, pl.ds(16 * q, 16)] = vals
          return ptr + 1

        ptr = lax.while_loop(w_cond, w_body, ptr)

        @pl.when(p + RING < ncols)
        def _():
          fetch(c_lo + p + RING, r)

      return ptr

    lax.fori_loop(0, (ncols + RING - 1) // RING, col_step, 0)
    pltpu.sync_copy(ostage, out_hbm.at[pl.ds(base // 8, cpw // 8)])

  return pl.kernel(
      body,
      out_type=jax.ShapeDtypeStruct((n // 8, 8, d), jnp.float32),
      mesh=mesh,
      scratch_types=[
          pltpu.VMEM((cpw + 16,), jnp.int32),
          pltpu.VMEM((RING, d, 128), jnp.float32),
          pltpu.VMEM((cpw // 8, 8, d), jnp.float32),
      ] + [pltpu.SemaphoreType.DMA] * RING,
      compiler_params=pltpu.CompilerParams(needs_layout_passes=False),
  )


@functools.cache
def _make_unpermute(d, n):
  cpw = n // NW
  mesh = plsc.VectorSubcoreMesh(core_axis_name="c", subcore_axis_name="s")

  def body(inv_hbm, src_hbm, out_hbm, idxv, ostage, sem):
    wid = lax.axis_index("s") * NC + lax.axis_index("c")
    base = wid * cpw

    pltpu.sync_copy(inv_hbm.at[pl.ds(base, cpw)], idxv)

    def fire(ch, carry):
      vec = idxv[pl.ds(ch * 16, 16)]
      for l in range(16):
        i = vec[l]
        pltpu.async_copy(
            src_hbm.at[i >> 3, i & 7], ostage.at[ch * 2 + l // 8, l % 8], sem)
      return carry

    lax.fori_loop(0, cpw // 16, fire, 0)

    def drain(k, carry):
      pltpu.make_async_copy(
          src_hbm.at[0, 0], ostage.at[k // 8, k % 8], sem).wait()
      return carry

    lax.fori_loop(0, cpw, drain, 0)
    pltpu.sync_copy(ostage, out_hbm.at[pl.ds(base // 8, cpw // 8)])

  return pl.kernel(
      body,
      out_type=jax.ShapeDtypeStruct((n // 8, 8, d), jnp.float32),
      mesh=mesh,
      scratch_types=[
          pltpu.VMEM((cpw,), jnp.int32),
          pltpu.VMEM((cpw // 8, 8, d), jnp.float32),
          pltpu.SemaphoreType.DMA,
      ],
  )


def kernel(user_indices, embedding_table):
  (n,) = user_indices.shape
  v, d = embedding_table.shape
  idx = user_indices.astype(jnp.int32)
  ar = jnp.arange(n, dtype=jnp.int32)
  idx_s, perm = lax.sort((idx, ar), num_keys=1)
  _, inv = lax.sort((perm, ar), num_keys=1)
  outs = _make_sorted_gather(v, d, n)(idx_s, embedding_table.T)
  out3 = _make_unpermute(d, n)(inv, outs)
  return out3.reshape(n, d)
